# baseline (speedup 1.0000x reference)
import jax
import jax.numpy as jnp
from jax import lax
from jax.experimental import pallas as pl
from jax.experimental.pallas import tpu as pltpu
from jax.experimental.pallas import tpu_sc as plsc

VOCAB = 100000
EMB = 64
EMBP = 128
HID = 128
TAGS = 64
B = 1024
T = 50
HALF = T // 2

NC = 2
NS = 16
NW = NC * NS
BT = B * T
ROWS_PER_W = BT // NW
CHUNK = 80
NCHUNK = ROWS_PER_W // CHUNK
NBUF = 8

PAD_ROWS = 4000


def _pad_body(src_ref, dst_ref):
    dst_ref[:, :EMB] = src_ref[...]
    dst_ref[:, EMB:] = jnp.zeros((PAD_ROWS, EMBP - EMB), jnp.float32)


def _pad_call(emb):
    return pl.pallas_call(
        _pad_body,
        grid=(VOCAB // PAD_ROWS,),
        in_specs=[pl.BlockSpec((PAD_ROWS, EMB), lambda i: (i, 0))],
        out_specs=pl.BlockSpec((PAD_ROWS, EMBP), lambda i: (i, 0)),
        out_shape=jax.ShapeDtypeStruct((VOCAB, EMBP), jnp.float32),
    )(emb)


def _gather_body(emb_hbm, idx_hbm, out_hbm, idx_v, rows_v, gsem, osem):
    wid = lax.axis_index("s") * NC + lax.axis_index("c")
    base = wid * ROWS_PER_W
    pltpu.sync_copy(idx_hbm.at[wid], idx_v)
    gathers = [None] * NCHUNK
    outs = [None] * NCHUNK
    for ci in range(NBUF):
        gathers[ci] = pltpu.async_copy(
            emb_hbm.at[idx_v.at[ci]], rows_v.at[ci], gsem
        )
    for ci in range(NCHUNK):
        gathers[ci].wait()
        outs[ci] = pltpu.async_copy(
            rows_v.at[ci % NBUF],
            out_hbm.at[pl.ds(base + ci * CHUNK, CHUNK)],
            osem,
        )
        nxt = ci + NBUF
        if nxt < NCHUNK:
            outs[ci].wait()
            gathers[nxt] = pltpu.async_copy(
                emb_hbm.at[idx_v.at[nxt]], rows_v.at[ci % NBUF], gsem
            )
    for ci in range(NCHUNK - NBUF, NCHUNK):
        outs[ci].wait()


def _gather_call(embp, idx):
    return pl.kernel(
        _gather_body,
        out_type=jax.ShapeDtypeStruct((BT, EMBP), jnp.float32),
        mesh=plsc.VectorSubcoreMesh(
            core_axis_name="c", subcore_axis_name="s",
            num_cores=NC, num_subcores=NS,
        ),
        scratch_types=[
            pltpu.VMEM((NCHUNK, CHUNK), jnp.int32),
            pltpu.VMEM((NBUF, CHUNK, EMBP), jnp.float32),
            pltpu.SemaphoreType.DMA,
            pltpu.SemaphoreType.DMA,
        ],
        compiler_params=pltpu.CompilerParams(use_tc_tiling_on_sc=False),
    )(embp, idx)


def _sig(x):
    return 0.5 * jnp.tanh(0.5 * x) + 0.5


def _lstm_step(first, e, h_ref, c_ref, wih_ref, whh_ref, bias_ref):
    hp = jnp.where(first, 0.0, h_ref[...])
    cp = jnp.where(first, 0.0, c_ref[...])
    g = jnp.dot(e, wih_ref[...], preferred_element_type=jnp.float32)
    g += jnp.dot(hp, whh_ref[...], preferred_element_type=jnp.float32)
    g += bias_ref[...]
    i = _sig(g[:, :HID])
    f = _sig(g[:, HID:2 * HID])
    gg = jnp.tanh(g[:, 2 * HID:3 * HID])
    o = _sig(g[:, 3 * HID:])
    c2 = f * cp + i * gg
    h2 = o * jnp.tanh(c2)
    h_ref[...] = h2
    c_ref[...] = c2
    return h2


def _bilstm_body(ef_ref, eb_ref, wih_f, whh_f, bias_f, wih_b, whh_b, bias_b,
                 fcw_f, fcw_b, fcb_ref, out_ref,
                 hf_ref, cf_ref, hb_ref, cb_ref, pb_store):
    t = pl.program_id(0)
    s = T - 1 - t
    first = t == 0

    h2f = _lstm_step(first, ef_ref[0], hf_ref, cf_ref, wih_f, whh_f, bias_f)
    h2b = _lstm_step(first, eb_ref[0], hb_ref, cb_ref, wih_b, whh_b, bias_b)
    pf = jnp.dot(h2f, fcw_f[...], preferred_element_type=jnp.float32)
    pb = jnp.dot(h2b, fcw_b[...], preferred_element_type=jnp.float32)

    @pl.when(t < HALF)
    def _():
        out_ref[pl.ds(t, 1)] = pf[None]
        pb_store[pl.ds(s - HALF, 1)] = pb[None]

    @pl.when(t >= HALF)
    def _():
        prior_pf = out_ref[pl.ds(s, 1)][0]
        out_ref[pl.ds(s, 1)] = _sig(prior_pf + pb + fcb_ref[...])[None]
        stored_pb = pb_store[pl.ds(t - HALF, 1)][0]
        out_ref[pl.ds(t, 1)] = _sig(pf + stored_pb + fcb_ref[...])[None]


def kernel(emb, w_ih_f, w_hh_f, b_ih_f, b_hh_f, w_ih_b, w_hh_b, b_ih_b,
           b_hh_b, fc_w, fc_b, x):
    embp = _pad_call(emb)
    idx = x.astype(jnp.int32).T.reshape(NW, NCHUNK, CHUNK)
    e_tb = _gather_call(embp, idx).reshape(T, B, EMBP)

    zpad = jnp.zeros((EMBP - EMB, 4 * HID), jnp.float32)
    wih_f_t = jnp.concatenate([w_ih_f.T, zpad], axis=0)
    whh_f_t = w_hh_f.T
    bias_f = (b_ih_f + b_hh_f).reshape(1, 4 * HID)
    wih_b_t = jnp.concatenate([w_ih_b.T, zpad], axis=0)
    whh_b_t = w_hh_b.T
    bias_b = (b_ih_b + b_hh_b).reshape(1, 4 * HID)
    fcw_t = fc_w.T
    fcw_f = fcw_t[:HID]
    fcw_b = fcw_t[HID:]
    fcb = fc_b.reshape(1, TAGS)

    def whole(shape):
        return pl.BlockSpec(shape, lambda t, _n=len(shape): (0,) * _n)

    out = pl.pallas_call(
        _bilstm_body,
        grid=(T,),
        in_specs=[
            pl.BlockSpec((1, B, EMBP), lambda t: (t, 0, 0)),
            pl.BlockSpec((1, B, EMBP), lambda t: (T - 1 - t, 0, 0)),
            whole((EMBP, 4 * HID)),
            whole((HID, 4 * HID)),
            whole((1, 4 * HID)),
            whole((EMBP, 4 * HID)),
            whole((HID, 4 * HID)),
            whole((1, 4 * HID)),
            whole((HID, TAGS)),
            whole((HID, TAGS)),
            whole((1, TAGS)),
        ],
        out_specs=whole((T, B, TAGS)),
        out_shape=jax.ShapeDtypeStruct((T, B, TAGS), jnp.float32),
        scratch_shapes=[
            pltpu.VMEM((B, HID), jnp.float32),
            pltpu.VMEM((B, HID), jnp.float32),
            pltpu.VMEM((B, HID), jnp.float32),
            pltpu.VMEM((B, HID), jnp.float32),
            pltpu.VMEM((HALF, B, TAGS), jnp.float32),
        ],
    )(e_tb, e_tb, wih_f_t, whh_f_t, bias_f, wih_b_t, whh_b_t, bias_b,
      fcw_f, fcw_b, fcb)

    return jnp.transpose(out, (1, 0, 2))


# speedup vs baseline: 1.0906x; 1.0906x over previous
import jax
import jax.numpy as jnp
from jax import lax
from jax.experimental import pallas as pl
from jax.experimental.pallas import tpu as pltpu
from jax.experimental.pallas import tpu_sc as plsc

VOCAB = 100000
EMB = 64
EMBP = 128
HID = 128
TAGS = 64
B = 1024
T = 50
HALF = T // 2

NC = 2
NS = 16
NW = NC * NS
BT = B * T
ROWS_PER_W = BT // NW
CHUNK = 80
NCHUNK = ROWS_PER_W // CHUNK
NBUF = 8

PAD_ROWS = 4000


def _pad_body(src_ref, dst_ref):
    dst_ref[:, :EMB] = src_ref[...]
    dst_ref[:, EMB:] = jnp.zeros((PAD_ROWS, EMBP - EMB), jnp.float32)


def _pad_call(emb):
    return pl.pallas_call(
        _pad_body,
        grid=(VOCAB // PAD_ROWS,),
        in_specs=[pl.BlockSpec((PAD_ROWS, EMB), lambda i: (i, 0))],
        out_specs=pl.BlockSpec((PAD_ROWS, EMBP), lambda i: (i, 0)),
        out_shape=jax.ShapeDtypeStruct((VOCAB, EMBP), jnp.float32),
    )(emb)


def _gather_body(emb_hbm, idx_hbm, out_hbm, idx_v, rows_v, gsem, osem):
    wid = lax.axis_index("s") * NC + lax.axis_index("c")
    base = wid * ROWS_PER_W
    pltpu.sync_copy(idx_hbm.at[wid], idx_v)
    gathers = [None] * NCHUNK
    outs = [None] * NCHUNK
    for ci in range(NBUF):
        gathers[ci] = pltpu.async_copy(
            emb_hbm.at[idx_v.at[ci]], rows_v.at[ci], gsem
        )
    for ci in range(NCHUNK):
        gathers[ci].wait()
        outs[ci] = pltpu.async_copy(
            rows_v.at[ci % NBUF],
            out_hbm.at[pl.ds(base + ci * CHUNK, CHUNK)],
            osem,
        )
        nxt = ci + NBUF
        if nxt < NCHUNK:
            outs[ci].wait()
            gathers[nxt] = pltpu.async_copy(
                emb_hbm.at[idx_v.at[nxt]], rows_v.at[ci % NBUF], gsem
            )
    for ci in range(NCHUNK - NBUF, NCHUNK):
        outs[ci].wait()


def _gather_call(embp, idx):
    return pl.kernel(
        _gather_body,
        out_type=jax.ShapeDtypeStruct((BT, EMBP), jnp.float32),
        mesh=plsc.VectorSubcoreMesh(
            core_axis_name="c", subcore_axis_name="s",
            num_cores=NC, num_subcores=NS,
        ),
        scratch_types=[
            pltpu.VMEM((NCHUNK, CHUNK), jnp.int32),
            pltpu.VMEM((NBUF, CHUNK, EMBP), jnp.float32),
            pltpu.SemaphoreType.DMA,
            pltpu.SemaphoreType.DMA,
        ],
        compiler_params=pltpu.CompilerParams(use_tc_tiling_on_sc=False),
    )(embp, idx)


def _sig(x):
    return 0.5 * jnp.tanh(0.5 * x) + 0.5


def _lstm_step(first, e, h_ref, c_ref, wih_ref, whh_ref, bias_ref):
    hp = jnp.where(first, 0.0, h_ref[...])
    cp = jnp.where(first, 0.0, c_ref[...])
    g = jnp.dot(e, wih_ref[...], preferred_element_type=jnp.float32)
    g += jnp.dot(hp, whh_ref[...], preferred_element_type=jnp.float32)
    g += bias_ref[...]
    i = _sig(g[:, :HID])
    f = _sig(g[:, HID:2 * HID])
    gg = jnp.tanh(g[:, 2 * HID:3 * HID])
    o = _sig(g[:, 3 * HID:])
    c2 = f * cp + i * gg
    h2 = o * jnp.tanh(c2)
    h_ref[...] = h2
    c_ref[...] = c2
    return h2


def _bilstm_body(ef_ref, eb_ref, wih_f, whh_f, bias_f, wih_b, whh_b, bias_b,
                 fcw_f, fcw_b, fcb_ref, out_ref,
                 hf_ref, cf_ref, hb_ref, cb_ref, pb_store):
    t = pl.program_id(0)
    s = T - 1 - t
    first = t == 0

    h2f = _lstm_step(first, ef_ref[0], hf_ref, cf_ref, wih_f, whh_f, bias_f)
    h2b = _lstm_step(first, eb_ref[0], hb_ref, cb_ref, wih_b, whh_b, bias_b)
    pf = jnp.dot(h2f, fcw_f[...], preferred_element_type=jnp.float32)
    pb = jnp.dot(h2b, fcw_b[...], preferred_element_type=jnp.float32)

    @pl.when(t < HALF)
    def _():
        out_ref[pl.ds(t, 1)] = pf[None]
        pb_store[pl.ds(s - HALF, 1)] = pb[None]

    @pl.when(t >= HALF)
    def _():
        prior_pf = out_ref[pl.ds(s, 1)][0]
        out_ref[pl.ds(s, 1)] = _sig(prior_pf + pb + fcb_ref[...])[None]
        stored_pb = pb_store[pl.ds(t - HALF, 1)][0]
        out_ref[pl.ds(t, 1)] = _sig(pf + stored_pb + fcb_ref[...])[None]


def kernel(emb, w_ih_f, w_hh_f, b_ih_f, b_hh_f, w_ih_b, w_hh_b, b_ih_b,
           b_hh_b, fc_w, fc_b, x):
    embp = jnp.pad(emb, ((0, 0), (0, EMBP - EMB)))
    idx = x.astype(jnp.int32).T.reshape(NW, NCHUNK, CHUNK)
    e_tb = _gather_call(embp, idx).reshape(T, B, EMBP)

    zpad = jnp.zeros((EMBP - EMB, 4 * HID), jnp.float32)
    wih_f_t = jnp.concatenate([w_ih_f.T, zpad], axis=0)
    whh_f_t = w_hh_f.T
    bias_f = (b_ih_f + b_hh_f).reshape(1, 4 * HID)
    wih_b_t = jnp.concatenate([w_ih_b.T, zpad], axis=0)
    whh_b_t = w_hh_b.T
    bias_b = (b_ih_b + b_hh_b).reshape(1, 4 * HID)
    fcw_t = fc_w.T
    fcw_f = fcw_t[:HID]
    fcw_b = fcw_t[HID:]
    fcb = fc_b.reshape(1, TAGS)

    def whole(shape):
        return pl.BlockSpec(shape, lambda t, _n=len(shape): (0,) * _n)

    out = pl.pallas_call(
        _bilstm_body,
        grid=(T,),
        in_specs=[
            pl.BlockSpec((1, B, EMBP), lambda t: (t, 0, 0)),
            pl.BlockSpec((1, B, EMBP), lambda t: (T - 1 - t, 0, 0)),
            whole((EMBP, 4 * HID)),
            whole((HID, 4 * HID)),
            whole((1, 4 * HID)),
            whole((EMBP, 4 * HID)),
            whole((HID, 4 * HID)),
            whole((1, 4 * HID)),
            whole((HID, TAGS)),
            whole((HID, TAGS)),
            whole((1, TAGS)),
        ],
        out_specs=whole((T, B, TAGS)),
        out_shape=jax.ShapeDtypeStruct((T, B, TAGS), jnp.float32),
        scratch_shapes=[
            pltpu.VMEM((B, HID), jnp.float32),
            pltpu.VMEM((B, HID), jnp.float32),
            pltpu.VMEM((B, HID), jnp.float32),
            pltpu.VMEM((B, HID), jnp.float32),
            pltpu.VMEM((HALF, B, TAGS), jnp.float32),
        ],
    )(e_tb, e_tb, wih_f_t, whh_f_t, bias_f, wih_b_t, whh_b_t, bias_b,
      fcw_f, fcw_b, fcb)

    return jnp.transpose(out, (1, 0, 2))


# speedup vs baseline: 1.1289x; 1.0352x over previous
import jax
import jax.numpy as jnp
from jax import lax
from jax.experimental import pallas as pl
from jax.experimental.pallas import tpu as pltpu
from jax.experimental.pallas import tpu_sc as plsc

VOCAB = 100000
EMB = 64
EMBP = 128
HID = 128
TAGS = 64
B = 1024
T = 50
HALF = T // 2

NC = 2
NS = 16
NW = NC * NS
BT = B * T
ROWS_PER_W = BT // NW
CHUNK = 80
NCHUNK = ROWS_PER_W // CHUNK
NBUF = 8

TBLK = 2048
TGRID = -(-VOCAB // TBLK)


def _tpad_body(et_ref, dst_ref):
    dst_ref[:, :EMB] = et_ref[...].T
    dst_ref[:, EMB:] = jnp.zeros((TBLK, EMBP - EMB), jnp.float32)


def _tpad_call(et):
    return pl.pallas_call(
        _tpad_body,
        grid=(TGRID,),
        in_specs=[pl.BlockSpec((EMB, TBLK), lambda i: (0, i))],
        out_specs=pl.BlockSpec((TBLK, EMBP), lambda i: (i, 0)),
        out_shape=jax.ShapeDtypeStruct((VOCAB, EMBP), jnp.float32),
    )(et)


def _gather_body(emb_hbm, idx_hbm, out_hbm, idx_v, rows_v, gsem, osem):
    wid = lax.axis_index("s") * NC + lax.axis_index("c")
    base = wid * ROWS_PER_W
    pltpu.sync_copy(idx_hbm.at[wid], idx_v)
    gathers = [None] * NCHUNK
    outs = [None] * NCHUNK
    for ci in range(NBUF):
        gathers[ci] = pltpu.async_copy(
            emb_hbm.at[idx_v.at[ci]], rows_v.at[ci], gsem
        )
    for ci in range(NCHUNK):
        gathers[ci].wait()
        outs[ci] = pltpu.async_copy(
            rows_v.at[ci % NBUF],
            out_hbm.at[pl.ds(base + ci * CHUNK, CHUNK)],
            osem,
        )
        nxt = ci + NBUF
        if nxt < NCHUNK:
            outs[ci].wait()
            gathers[nxt] = pltpu.async_copy(
                emb_hbm.at[idx_v.at[nxt]], rows_v.at[ci % NBUF], gsem
            )
    for ci in range(NCHUNK - NBUF, NCHUNK):
        outs[ci].wait()


def _gather_call(embp, idx):
    return pl.kernel(
        _gather_body,
        out_type=jax.ShapeDtypeStruct((BT, EMBP), jnp.float32),
        mesh=plsc.VectorSubcoreMesh(
            core_axis_name="c", subcore_axis_name="s",
            num_cores=NC, num_subcores=NS,
        ),
        scratch_types=[
            pltpu.VMEM((NCHUNK, CHUNK), jnp.int32),
            pltpu.VMEM((NBUF, CHUNK, EMBP), jnp.float32),
            pltpu.SemaphoreType.DMA,
            pltpu.SemaphoreType.DMA,
        ],
        compiler_params=pltpu.CompilerParams(use_tc_tiling_on_sc=False),
    )(embp, idx)


def _sig(x):
    return 0.5 * jnp.tanh(0.5 * x) + 0.5


def _lstm_step(first, e, h_ref, c_ref, wih_ref, whh_ref, bias_ref):
    hp = jnp.where(first, 0.0, h_ref[...])
    cp = jnp.where(first, 0.0, c_ref[...])
    g = jnp.dot(e, wih_ref[...], preferred_element_type=jnp.float32)
    g += jnp.dot(hp, whh_ref[...], preferred_element_type=jnp.float32)
    g += bias_ref[...]
    i = _sig(g[:, :HID])
    f = _sig(g[:, HID:2 * HID])
    gg = jnp.tanh(g[:, 2 * HID:3 * HID])
    o = _sig(g[:, 3 * HID:])
    c2 = f * cp + i * gg
    h2 = o * jnp.tanh(c2)
    h_ref[...] = h2
    c_ref[...] = c2
    return h2


def _bilstm_body(ef_ref, eb_ref, wih_f, whh_f, bias_f, wih_b, whh_b, bias_b,
                 fcw_f, fcw_b, fcb_ref, out_ref,
                 hf_ref, cf_ref, hb_ref, cb_ref, pb_store):
    t = pl.program_id(0)
    s = T - 1 - t
    first = t == 0

    h2f = _lstm_step(first, ef_ref[0], hf_ref, cf_ref, wih_f, whh_f, bias_f)
    h2b = _lstm_step(first, eb_ref[0], hb_ref, cb_ref, wih_b, whh_b, bias_b)
    pf = jnp.dot(h2f, fcw_f[...], preferred_element_type=jnp.float32)
    pb = jnp.dot(h2b, fcw_b[...], preferred_element_type=jnp.float32)

    @pl.when(t < HALF)
    def _():
        out_ref[pl.ds(t, 1)] = pf[None]
        pb_store[pl.ds(s - HALF, 1)] = pb[None]

    @pl.when(t >= HALF)
    def _():
        prior_pf = out_ref[pl.ds(s, 1)][0]
        out_ref[pl.ds(s, 1)] = _sig(prior_pf + pb + fcb_ref[...])[None]
        stored_pb = pb_store[pl.ds(t - HALF, 1)][0]
        out_ref[pl.ds(t, 1)] = _sig(pf + stored_pb + fcb_ref[...])[None]


def kernel(emb, w_ih_f, w_hh_f, b_ih_f, b_hh_f, w_ih_b, w_hh_b, b_ih_b,
           b_hh_b, fc_w, fc_b, x):
    embp = _tpad_call(emb.T)
    idx = x.astype(jnp.int32).T.reshape(NW, NCHUNK, CHUNK)
    e_tb = _gather_call(embp, idx).reshape(T, B, EMBP)

    zpad = jnp.zeros((EMBP - EMB, 4 * HID), jnp.float32)
    wih_f_t = jnp.concatenate([w_ih_f.T, zpad], axis=0)
    whh_f_t = w_hh_f.T
    bias_f = (b_ih_f + b_hh_f).reshape(1, 4 * HID)
    wih_b_t = jnp.concatenate([w_ih_b.T, zpad], axis=0)
    whh_b_t = w_hh_b.T
    bias_b = (b_ih_b + b_hh_b).reshape(1, 4 * HID)
    fcw_t = fc_w.T
    fcw_f = fcw_t[:HID]
    fcw_b = fcw_t[HID:]
    fcb = fc_b.reshape(1, TAGS)

    def whole(shape):
        return pl.BlockSpec(shape, lambda t, _n=len(shape): (0,) * _n)

    out = pl.pallas_call(
        _bilstm_body,
        grid=(T,),
        in_specs=[
            pl.BlockSpec((1, B, EMBP), lambda t: (t, 0, 0)),
            pl.BlockSpec((1, B, EMBP), lambda t: (T - 1 - t, 0, 0)),
            whole((EMBP, 4 * HID)),
            whole((HID, 4 * HID)),
            whole((1, 4 * HID)),
            whole((EMBP, 4 * HID)),
            whole((HID, 4 * HID)),
            whole((1, 4 * HID)),
            whole((HID, TAGS)),
            whole((HID, TAGS)),
            whole((1, TAGS)),
        ],
        out_specs=whole((T, B, TAGS)),
        out_shape=jax.ShapeDtypeStruct((T, B, TAGS), jnp.float32),
        scratch_shapes=[
            pltpu.VMEM((B, HID), jnp.float32),
            pltpu.VMEM((B, HID), jnp.float32),
            pltpu.VMEM((B, HID), jnp.float32),
            pltpu.VMEM((B, HID), jnp.float32),
            pltpu.VMEM((HALF, B, TAGS), jnp.float32),
        ],
    )(e_tb, e_tb, wih_f_t, whh_f_t, bias_f, wih_b_t, whh_b_t, bias_b,
      fcw_f, fcw_b, fcb)

    return jnp.transpose(out, (1, 0, 2))


# speedup vs baseline: 1.2056x; 1.0679x over previous
import jax
import jax.numpy as jnp
from jax import lax
from jax.experimental import pallas as pl
from jax.experimental.pallas import tpu as pltpu
from jax.experimental.pallas import tpu_sc as plsc

VOCAB = 100000
EMB = 64
EMBP = 128
HID = 128
TAGS = 64
B = 1024
T = 50
HALF = T // 2

NC = 2
NS = 16
NW = NC * NS
BT = B * T
ROWS_PER_W = BT // NW
CHUNK = 80
NCHUNK = ROWS_PER_W // CHUNK
NBUF = 8

TBLK = 4096
TGRID = -(-VOCAB // TBLK)


def _tpad_body(et_ref, dst_ref):
    dst_ref[:, :EMB] = et_ref[...].T
    dst_ref[:, EMB:] = jnp.zeros((TBLK, EMBP - EMB), jnp.float32)


def _tpad_call(et):
    return pl.pallas_call(
        _tpad_body,
        grid=(TGRID,),
        in_specs=[pl.BlockSpec((EMB, TBLK), lambda i: (0, i))],
        out_specs=pl.BlockSpec((TBLK, EMBP), lambda i: (i, 0)),
        out_shape=jax.ShapeDtypeStruct((VOCAB, EMBP), jnp.float32),
    )(et)


def _gather_body(emb_hbm, idx_hbm, out_hbm, idx_v, rows_v, gsem, osem):
    wid = lax.axis_index("s") * NC + lax.axis_index("c")
    base = wid * ROWS_PER_W
    pltpu.sync_copy(idx_hbm.at[wid], idx_v)
    gathers = [None] * NCHUNK
    outs = [None] * NCHUNK
    for ci in range(NBUF):
        gathers[ci] = pltpu.async_copy(
            emb_hbm.at[idx_v.at[ci]], rows_v.at[ci], gsem
        )
    for ci in range(NCHUNK):
        gathers[ci].wait()
        outs[ci] = pltpu.async_copy(
            rows_v.at[ci % NBUF],
            out_hbm.at[pl.ds(base + ci * CHUNK, CHUNK)],
            osem,
        )
        nxt = ci + NBUF
        if nxt < NCHUNK:
            outs[ci].wait()
            gathers[nxt] = pltpu.async_copy(
                emb_hbm.at[idx_v.at[nxt]], rows_v.at[ci % NBUF], gsem
            )
    for ci in range(NCHUNK - NBUF, NCHUNK):
        outs[ci].wait()


def _gather_call(embp, idx):
    return pl.kernel(
        _gather_body,
        out_type=jax.ShapeDtypeStruct((BT, EMBP), jnp.float32),
        mesh=plsc.VectorSubcoreMesh(
            core_axis_name="c", subcore_axis_name="s",
            num_cores=NC, num_subcores=NS,
        ),
        scratch_types=[
            pltpu.VMEM((NCHUNK, CHUNK), jnp.int32),
            pltpu.VMEM((NBUF, CHUNK, EMBP), jnp.float32),
            pltpu.SemaphoreType.DMA,
            pltpu.SemaphoreType.DMA,
        ],
        compiler_params=pltpu.CompilerParams(use_tc_tiling_on_sc=False),
    )(embp, idx)


def _sig(x):
    return 0.5 * jnp.tanh(0.5 * x) + 0.5


def _lstm_step(first, e, h_ref, c_ref, wih_ref, whh_ref, bias_ref):
    hp = jnp.where(first, 0.0, h_ref[...])
    cp = jnp.where(first, 0.0, c_ref[...])
    g = jnp.dot(e, wih_ref[...], preferred_element_type=jnp.float32)
    g += jnp.dot(hp, whh_ref[...], preferred_element_type=jnp.float32)
    g += bias_ref[...]
    i = _sig(g[:, :HID])
    f = _sig(g[:, HID:2 * HID])
    gg = jnp.tanh(g[:, 2 * HID:3 * HID])
    o = _sig(g[:, 3 * HID:])
    c2 = f * cp + i * gg
    h2 = o * jnp.tanh(c2)
    h_ref[...] = h2
    c_ref[...] = c2
    return h2


def _bilstm_body(ef_ref, eb_ref, wih_f, whh_f, bias_f, wih_b, whh_b, bias_b,
                 fcw_f, fcw_b, fcb_ref, out_ref,
                 hf_ref, cf_ref, hb_ref, cb_ref, pb_store):
    t = pl.program_id(0)
    s = T - 1 - t
    first = t == 0

    h2f = _lstm_step(first, ef_ref[0], hf_ref, cf_ref, wih_f, whh_f, bias_f)
    h2b = _lstm_step(first, eb_ref[0], hb_ref, cb_ref, wih_b, whh_b, bias_b)
    pf = jnp.dot(h2f, fcw_f[...], preferred_element_type=jnp.float32)
    pb = jnp.dot(h2b, fcw_b[...], preferred_element_type=jnp.float32)

    @pl.when(t < HALF)
    def _():
        out_ref[pl.ds(t, 1)] = pf[None]
        pb_store[pl.ds(s - HALF, 1)] = pb[None]

    @pl.when(t >= HALF)
    def _():
        prior_pf = out_ref[pl.ds(s, 1)][0]
        out_ref[pl.ds(s, 1)] = _sig(prior_pf + pb + fcb_ref[...])[None]
        stored_pb = pb_store[pl.ds(t - HALF, 1)][0]
        out_ref[pl.ds(t, 1)] = _sig(pf + stored_pb + fcb_ref[...])[None]


def kernel(emb, w_ih_f, w_hh_f, b_ih_f, b_hh_f, w_ih_b, w_hh_b, b_ih_b,
           b_hh_b, fc_w, fc_b, x):
    embp = _tpad_call(emb.T)
    idx = x.astype(jnp.int32).T.reshape(NW, NCHUNK, CHUNK)
    e_tb = _gather_call(embp, idx).reshape(T, B, EMBP)

    zpad = jnp.zeros((EMBP - EMB, 4 * HID), jnp.float32)
    wih_f_t = jnp.concatenate([w_ih_f.T, zpad], axis=0)
    whh_f_t = w_hh_f.T
    bias_f = (b_ih_f + b_hh_f).reshape(1, 4 * HID)
    wih_b_t = jnp.concatenate([w_ih_b.T, zpad], axis=0)
    whh_b_t = w_hh_b.T
    bias_b = (b_ih_b + b_hh_b).reshape(1, 4 * HID)
    fcw_t = fc_w.T
    fcw_f = fcw_t[:HID]
    fcw_b = fcw_t[HID:]
    fcb = fc_b.reshape(1, TAGS)

    def whole(shape):
        return pl.BlockSpec(shape, lambda t, _n=len(shape): (0,) * _n)

    out = pl.pallas_call(
        _bilstm_body,
        grid=(T,),
        in_specs=[
            pl.BlockSpec((1, B, EMBP), lambda t: (t, 0, 0)),
            pl.BlockSpec((1, B, EMBP), lambda t: (T - 1 - t, 0, 0)),
            whole((EMBP, 4 * HID)),
            whole((HID, 4 * HID)),
            whole((1, 4 * HID)),
            whole((EMBP, 4 * HID)),
            whole((HID, 4 * HID)),
            whole((1, 4 * HID)),
            whole((HID, TAGS)),
            whole((HID, TAGS)),
            whole((1, TAGS)),
        ],
        out_specs=whole((T, B, TAGS)),
        out_shape=jax.ShapeDtypeStruct((T, B, TAGS), jnp.float32),
        scratch_shapes=[
            pltpu.VMEM((B, HID), jnp.float32),
            pltpu.VMEM((B, HID), jnp.float32),
            pltpu.VMEM((B, HID), jnp.float32),
            pltpu.VMEM((B, HID), jnp.float32),
            pltpu.VMEM((HALF, B, TAGS), jnp.float32),
        ],
    )(e_tb, e_tb, wih_f_t, whh_f_t, bias_f, wih_b_t, whh_b_t, bias_b,
      fcw_f, fcw_b, fcb)

    return jnp.transpose(out, (1, 0, 2))


# speedup vs baseline: 1.2599x; 1.0451x over previous
import jax
import jax.numpy as jnp
from jax import lax
from jax.experimental import pallas as pl
from jax.experimental.pallas import tpu as pltpu
from jax.experimental.pallas import tpu_sc as plsc

VOCAB = 100000
EMB = 64
EMBP = 128
HID = 128
TAGS = 64
B = 1024
T = 50
HALF = T // 2

NC = 2
NS = 16
NW = NC * NS
BT = B * T
ROWS_PER_W = BT // NW
CHUNK = 80
NCHUNK = ROWS_PER_W // CHUNK
NBUF = 8

TBLK = 8192
TGRID = -(-VOCAB // TBLK)


def _tpad_body(et_ref, dst_ref):
    dst_ref[:, :EMB] = et_ref[...].T
    dst_ref[:, EMB:] = jnp.zeros((TBLK, EMBP - EMB), jnp.float32)


def _tpad_call(et):
    return pl.pallas_call(
        _tpad_body,
        grid=(TGRID,),
        in_specs=[pl.BlockSpec((EMB, TBLK), lambda i: (0, i))],
        out_specs=pl.BlockSpec((TBLK, EMBP), lambda i: (i, 0)),
        out_shape=jax.ShapeDtypeStruct((VOCAB, EMBP), jnp.float32),
    )(et)


def _gather_body(emb_hbm, idx_hbm, out_hbm, idx_v, rows_v, gsem, osem):
    wid = lax.axis_index("s") * NC + lax.axis_index("c")
    base = wid * ROWS_PER_W
    pltpu.sync_copy(idx_hbm.at[wid], idx_v)
    gathers = [None] * NCHUNK
    outs = [None] * NCHUNK
    for ci in range(NBUF):
        gathers[ci] = pltpu.async_copy(
            emb_hbm.at[idx_v.at[ci]], rows_v.at[ci], gsem
        )
    for ci in range(NCHUNK):
        gathers[ci].wait()
        outs[ci] = pltpu.async_copy(
            rows_v.at[ci % NBUF],
            out_hbm.at[pl.ds(base + ci * CHUNK, CHUNK)],
            osem,
        )
        nxt = ci + NBUF
        if nxt < NCHUNK:
            outs[ci].wait()
            gathers[nxt] = pltpu.async_copy(
                emb_hbm.at[idx_v.at[nxt]], rows_v.at[ci % NBUF], gsem
            )
    for ci in range(NCHUNK - NBUF, NCHUNK):
        outs[ci].wait()


def _gather_call(embp, idx):
    return pl.kernel(
        _gather_body,
        out_type=jax.ShapeDtypeStruct((BT, EMBP), jnp.float32),
        mesh=plsc.VectorSubcoreMesh(
            core_axis_name="c", subcore_axis_name="s",
            num_cores=NC, num_subcores=NS,
        ),
        scratch_types=[
            pltpu.VMEM((NCHUNK, CHUNK), jnp.int32),
            pltpu.VMEM((NBUF, CHUNK, EMBP), jnp.float32),
            pltpu.SemaphoreType.DMA,
            pltpu.SemaphoreType.DMA,
        ],
        compiler_params=pltpu.CompilerParams(use_tc_tiling_on_sc=False),
    )(embp, idx)


def _sig(x):
    return 0.5 * jnp.tanh(0.5 * x) + 0.5


def _lstm_step(first, e, h_ref, c_ref, wih_ref, whh_ref, bias_ref):
    hp = jnp.where(first, 0.0, h_ref[...])
    cp = jnp.where(first, 0.0, c_ref[...])
    g = jnp.dot(e, wih_ref[...], preferred_element_type=jnp.float32)
    g += jnp.dot(hp, whh_ref[...], preferred_element_type=jnp.float32)
    g += bias_ref[...]
    i = _sig(g[:, :HID])
    f = _sig(g[:, HID:2 * HID])
    gg = jnp.tanh(g[:, 2 * HID:3 * HID])
    o = _sig(g[:, 3 * HID:])
    c2 = f * cp + i * gg
    h2 = o * jnp.tanh(c2)
    h_ref[...] = h2
    c_ref[...] = c2
    return h2


def _bilstm_body(ef_ref, eb_ref, wih_f, whh_f, bias_f, wih_b, whh_b, bias_b,
                 fcw_f, fcw_b, fcb_ref, out_ref,
                 hf_ref, cf_ref, hb_ref, cb_ref, pb_store):
    t = pl.program_id(0)
    s = T - 1 - t
    first = t == 0

    h2f = _lstm_step(first, ef_ref[0], hf_ref, cf_ref, wih_f, whh_f, bias_f)
    h2b = _lstm_step(first, eb_ref[0], hb_ref, cb_ref, wih_b, whh_b, bias_b)
    pf = jnp.dot(h2f, fcw_f[...], preferred_element_type=jnp.float32)
    pb = jnp.dot(h2b, fcw_b[...], preferred_element_type=jnp.float32)

    @pl.when(t < HALF)
    def _():
        out_ref[pl.ds(t, 1)] = pf[None]
        pb_store[pl.ds(s - HALF, 1)] = pb[None]

    @pl.when(t >= HALF)
    def _():
        prior_pf = out_ref[pl.ds(s, 1)][0]
        out_ref[pl.ds(s, 1)] = _sig(prior_pf + pb + fcb_ref[...])[None]
        stored_pb = pb_store[pl.ds(t - HALF, 1)][0]
        out_ref[pl.ds(t, 1)] = _sig(pf + stored_pb + fcb_ref[...])[None]


def kernel(emb, w_ih_f, w_hh_f, b_ih_f, b_hh_f, w_ih_b, w_hh_b, b_ih_b,
           b_hh_b, fc_w, fc_b, x):
    embp = _tpad_call(emb.T)
    idx = x.astype(jnp.int32).T.reshape(NW, NCHUNK, CHUNK)
    e_tb = _gather_call(embp, idx).reshape(T, B, EMBP)

    zpad = jnp.zeros((EMBP - EMB, 4 * HID), jnp.float32)
    wih_f_t = jnp.concatenate([w_ih_f.T, zpad], axis=0)
    whh_f_t = w_hh_f.T
    bias_f = (b_ih_f + b_hh_f).reshape(1, 4 * HID)
    wih_b_t = jnp.concatenate([w_ih_b.T, zpad], axis=0)
    whh_b_t = w_hh_b.T
    bias_b = (b_ih_b + b_hh_b).reshape(1, 4 * HID)
    fcw_t = fc_w.T
    fcw_f = fcw_t[:HID]
    fcw_b = fcw_t[HID:]
    fcb = fc_b.reshape(1, TAGS)

    def whole(shape):
        return pl.BlockSpec(shape, lambda t, _n=len(shape): (0,) * _n)

    out = pl.pallas_call(
        _bilstm_body,
        grid=(T,),
        in_specs=[
            pl.BlockSpec((1, B, EMBP), lambda t: (t, 0, 0)),
            pl.BlockSpec((1, B, EMBP), lambda t: (T - 1 - t, 0, 0)),
            whole((EMBP, 4 * HID)),
            whole((HID, 4 * HID)),
            whole((1, 4 * HID)),
            whole((EMBP, 4 * HID)),
            whole((HID, 4 * HID)),
            whole((1, 4 * HID)),
            whole((HID, TAGS)),
            whole((HID, TAGS)),
            whole((1, TAGS)),
        ],
        out_specs=whole((T, B, TAGS)),
        out_shape=jax.ShapeDtypeStruct((T, B, TAGS), jnp.float32),
        scratch_shapes=[
            pltpu.VMEM((B, HID), jnp.float32),
            pltpu.VMEM((B, HID), jnp.float32),
            pltpu.VMEM((B, HID), jnp.float32),
            pltpu.VMEM((B, HID), jnp.float32),
            pltpu.VMEM((HALF, B, TAGS), jnp.float32),
        ],
    )(e_tb, e_tb, wih_f_t, whh_f_t, bias_f, wih_b_t, whh_b_t, bias_b,
      fcw_f, fcw_b, fcb)

    return jnp.transpose(out, (1, 0, 2))


# speedup vs baseline: 1.2645x; 1.0037x over previous
import jax
import jax.numpy as jnp
from jax import lax
from jax.experimental import pallas as pl
from jax.experimental.pallas import tpu as pltpu
from jax.experimental.pallas import tpu_sc as plsc

VOCAB = 100000
EMB = 64
EMBP = 128
HID = 128
TAGS = 64
B = 1024
T = 50
HALF = T // 2

NC = 2
NS = 16
NW = NC * NS
BT = B * T
ROWS_PER_W = BT // NW
CHUNK = 80
NCHUNK = ROWS_PER_W // CHUNK
NBUF = 8

TBLK = 12800
TGRID = -(-VOCAB // TBLK)


def _tpad_body(et_ref, dst_ref):
    dst_ref[:, :EMB] = et_ref[...].T
    dst_ref[:, EMB:] = jnp.zeros((TBLK, EMBP - EMB), jnp.float32)


def _tpad_call(et):
    return pl.pallas_call(
        _tpad_body,
        grid=(TGRID,),
        in_specs=[pl.BlockSpec((EMB, TBLK), lambda i: (0, i))],
        out_specs=pl.BlockSpec((TBLK, EMBP), lambda i: (i, 0)),
        out_shape=jax.ShapeDtypeStruct((VOCAB, EMBP), jnp.float32),
    )(et)


def _gather_body(emb_hbm, idx_hbm, out_hbm, idx_v, rows_v, gsem, osem):
    wid = lax.axis_index("s") * NC + lax.axis_index("c")
    base = wid * ROWS_PER_W
    pltpu.sync_copy(idx_hbm.at[wid], idx_v)
    gathers = [None] * NCHUNK
    outs = [None] * NCHUNK
    for ci in range(NBUF):
        gathers[ci] = pltpu.async_copy(
            emb_hbm.at[idx_v.at[ci]], rows_v.at[ci], gsem
        )
    for ci in range(NCHUNK):
        gathers[ci].wait()
        outs[ci] = pltpu.async_copy(
            rows_v.at[ci % NBUF],
            out_hbm.at[pl.ds(base + ci * CHUNK, CHUNK)],
            osem,
        )
        nxt = ci + NBUF
        if nxt < NCHUNK:
            outs[ci].wait()
            gathers[nxt] = pltpu.async_copy(
                emb_hbm.at[idx_v.at[nxt]], rows_v.at[ci % NBUF], gsem
            )
    for ci in range(NCHUNK - NBUF, NCHUNK):
        outs[ci].wait()


def _gather_call(embp, idx):
    return pl.kernel(
        _gather_body,
        out_type=jax.ShapeDtypeStruct((BT, EMBP), jnp.float32),
        mesh=plsc.VectorSubcoreMesh(
            core_axis_name="c", subcore_axis_name="s",
            num_cores=NC, num_subcores=NS,
        ),
        scratch_types=[
            pltpu.VMEM((NCHUNK, CHUNK), jnp.int32),
            pltpu.VMEM((NBUF, CHUNK, EMBP), jnp.float32),
            pltpu.SemaphoreType.DMA,
            pltpu.SemaphoreType.DMA,
        ],
        compiler_params=pltpu.CompilerParams(use_tc_tiling_on_sc=False),
    )(embp, idx)


def _sig(x):
    return 0.5 * jnp.tanh(0.5 * x) + 0.5


def _lstm_step(first, e, h_ref, c_ref, wih_ref, whh_ref, bias_ref):
    hp = jnp.where(first, 0.0, h_ref[...])
    cp = jnp.where(first, 0.0, c_ref[...])
    g = jnp.dot(e, wih_ref[...], preferred_element_type=jnp.float32)
    g += jnp.dot(hp, whh_ref[...], preferred_element_type=jnp.float32)
    g += bias_ref[...]
    i = _sig(g[:, :HID])
    f = _sig(g[:, HID:2 * HID])
    gg = jnp.tanh(g[:, 2 * HID:3 * HID])
    o = _sig(g[:, 3 * HID:])
    c2 = f * cp + i * gg
    h2 = o * jnp.tanh(c2)
    h_ref[...] = h2
    c_ref[...] = c2
    return h2


def _bilstm_body(ef_ref, eb_ref, wih_f, whh_f, bias_f, wih_b, whh_b, bias_b,
                 fcw_f, fcw_b, fcb_ref, out_ref,
                 hf_ref, cf_ref, hb_ref, cb_ref, pb_store):
    t = pl.program_id(0)
    s = T - 1 - t
    first = t == 0

    h2f = _lstm_step(first, ef_ref[0], hf_ref, cf_ref, wih_f, whh_f, bias_f)
    h2b = _lstm_step(first, eb_ref[0], hb_ref, cb_ref, wih_b, whh_b, bias_b)
    pf = jnp.dot(h2f, fcw_f[...], preferred_element_type=jnp.float32)
    pb = jnp.dot(h2b, fcw_b[...], preferred_element_type=jnp.float32)

    @pl.when(t < HALF)
    def _():
        out_ref[pl.ds(t, 1)] = pf[None]
        pb_store[pl.ds(s - HALF, 1)] = pb[None]

    @pl.when(t >= HALF)
    def _():
        prior_pf = out_ref[pl.ds(s, 1)][0]
        out_ref[pl.ds(s, 1)] = _sig(prior_pf + pb + fcb_ref[...])[None]
        stored_pb = pb_store[pl.ds(t - HALF, 1)][0]
        out_ref[pl.ds(t, 1)] = _sig(pf + stored_pb + fcb_ref[...])[None]


def kernel(emb, w_ih_f, w_hh_f, b_ih_f, b_hh_f, w_ih_b, w_hh_b, b_ih_b,
           b_hh_b, fc_w, fc_b, x):
    embp = _tpad_call(emb.T)
    idx = x.astype(jnp.int32).T.reshape(NW, NCHUNK, CHUNK)
    e_tb = _gather_call(embp, idx).reshape(T, B, EMBP)

    zpad = jnp.zeros((EMBP - EMB, 4 * HID), jnp.float32)
    wih_f_t = jnp.concatenate([w_ih_f.T, zpad], axis=0)
    whh_f_t = w_hh_f.T
    bias_f = (b_ih_f + b_hh_f).reshape(1, 4 * HID)
    wih_b_t = jnp.concatenate([w_ih_b.T, zpad], axis=0)
    whh_b_t = w_hh_b.T
    bias_b = (b_ih_b + b_hh_b).reshape(1, 4 * HID)
    fcw_t = fc_w.T
    fcw_f = fcw_t[:HID]
    fcw_b = fcw_t[HID:]
    fcb = fc_b.reshape(1, TAGS)

    def whole(shape):
        return pl.BlockSpec(shape, lambda t, _n=len(shape): (0,) * _n)

    out = pl.pallas_call(
        _bilstm_body,
        grid=(T,),
        in_specs=[
            pl.BlockSpec((1, B, EMBP), lambda t: (t, 0, 0)),
            pl.BlockSpec((1, B, EMBP), lambda t: (T - 1 - t, 0, 0)),
            whole((EMBP, 4 * HID)),
            whole((HID, 4 * HID)),
            whole((1, 4 * HID)),
            whole((EMBP, 4 * HID)),
            whole((HID, 4 * HID)),
            whole((1, 4 * HID)),
            whole((HID, TAGS)),
            whole((HID, TAGS)),
            whole((1, TAGS)),
        ],
        out_specs=whole((T, B, TAGS)),
        out_shape=jax.ShapeDtypeStruct((T, B, TAGS), jnp.float32),
        scratch_shapes=[
            pltpu.VMEM((B, HID), jnp.float32),
            pltpu.VMEM((B, HID), jnp.float32),
            pltpu.VMEM((B, HID), jnp.float32),
            pltpu.VMEM((B, HID), jnp.float32),
            pltpu.VMEM((HALF, B, TAGS), jnp.float32),
        ],
    )(e_tb, e_tb, wih_f_t, whh_f_t, bias_f, wih_b_t, whh_b_t, bias_b,
      fcw_f, fcw_b, fcb)

    return jnp.transpose(out, (1, 0, 2))


# speedup vs baseline: 1.2649x; 1.0003x over previous
import jax
import jax.numpy as jnp
from jax import lax
from jax.experimental import pallas as pl
from jax.experimental.pallas import tpu as pltpu
from jax.experimental.pallas import tpu_sc as plsc

VOCAB = 100000
EMB = 64
EMBP = 128
HID = 128
TAGS = 64
B = 1024
T = 50
HALF = T // 2

NC = 2
NS = 16
NW = NC * NS
BT = B * T
ROWS_PER_W = BT // NW
CHUNK = 80
NCHUNK = ROWS_PER_W // CHUNK
NBUF = 10

TBLK = 12800
TGRID = -(-VOCAB // TBLK)


def _tpad_body(et_ref, dst_ref):
    dst_ref[:, :EMB] = et_ref[...].T
    dst_ref[:, EMB:] = jnp.zeros((TBLK, EMBP - EMB), jnp.float32)


def _tpad_call(et):
    return pl.pallas_call(
        _tpad_body,
        grid=(TGRID,),
        in_specs=[pl.BlockSpec((EMB, TBLK), lambda i: (0, i))],
        out_specs=pl.BlockSpec((TBLK, EMBP), lambda i: (i, 0)),
        out_shape=jax.ShapeDtypeStruct((VOCAB, EMBP), jnp.float32),
    )(et)


def _gather_body(emb_hbm, idx_hbm, out_hbm, idx_v, rows_v, gsem, osem):
    wid = lax.axis_index("s") * NC + lax.axis_index("c")
    base = wid * ROWS_PER_W
    pltpu.sync_copy(idx_hbm.at[wid], idx_v)
    gathers = [None] * NCHUNK
    outs = [None] * NCHUNK
    for ci in range(NBUF):
        gathers[ci] = pltpu.async_copy(
            emb_hbm.at[idx_v.at[ci]], rows_v.at[ci], gsem
        )
    for ci in range(NCHUNK):
        gathers[ci].wait()
        outs[ci] = pltpu.async_copy(
            rows_v.at[ci % NBUF],
            out_hbm.at[pl.ds(base + ci * CHUNK, CHUNK)],
            osem,
        )
        nxt = ci + NBUF
        if nxt < NCHUNK:
            outs[ci].wait()
            gathers[nxt] = pltpu.async_copy(
                emb_hbm.at[idx_v.at[nxt]], rows_v.at[ci % NBUF], gsem
            )
    for ci in range(NCHUNK - NBUF, NCHUNK):
        outs[ci].wait()


def _gather_call(embp, idx):
    return pl.kernel(
        _gather_body,
        out_type=jax.ShapeDtypeStruct((BT, EMBP), jnp.float32),
        mesh=plsc.VectorSubcoreMesh(
            core_axis_name="c", subcore_axis_name="s",
            num_cores=NC, num_subcores=NS,
        ),
        scratch_types=[
            pltpu.VMEM((NCHUNK, CHUNK), jnp.int32),
            pltpu.VMEM((NBUF, CHUNK, EMBP), jnp.float32),
            pltpu.SemaphoreType.DMA,
            pltpu.SemaphoreType.DMA,
        ],
        compiler_params=pltpu.CompilerParams(use_tc_tiling_on_sc=False),
    )(embp, idx)


def _sig(x):
    return 0.5 * jnp.tanh(0.5 * x) + 0.5


def _lstm_step(first, e, h_ref, c_ref, wih_ref, whh_ref, bias_ref):
    hp = jnp.where(first, 0.0, h_ref[...])
    cp = jnp.where(first, 0.0, c_ref[...])
    g = jnp.dot(e, wih_ref[...], preferred_element_type=jnp.float32)
    g += jnp.dot(hp, whh_ref[...], preferred_element_type=jnp.float32)
    g += bias_ref[...]
    i = _sig(g[:, :HID])
    f = _sig(g[:, HID:2 * HID])
    gg = jnp.tanh(g[:, 2 * HID:3 * HID])
    o = _sig(g[:, 3 * HID:])
    c2 = f * cp + i * gg
    h2 = o * jnp.tanh(c2)
    h_ref[...] = h2
    c_ref[...] = c2
    return h2


def _bilstm_body(ef_ref, eb_ref, wih_f, whh_f, bias_f, wih_b, whh_b, bias_b,
                 fcw_f, fcw_b, fcb_ref, out_ref,
                 hf_ref, cf_ref, hb_ref, cb_ref, pb_store):
    t = pl.program_id(0)
    s = T - 1 - t
    first = t == 0

    h2f = _lstm_step(first, ef_ref[0], hf_ref, cf_ref, wih_f, whh_f, bias_f)
    h2b = _lstm_step(first, eb_ref[0], hb_ref, cb_ref, wih_b, whh_b, bias_b)
    pf = jnp.dot(h2f, fcw_f[...], preferred_element_type=jnp.float32)
    pb = jnp.dot(h2b, fcw_b[...], preferred_element_type=jnp.float32)

    @pl.when(t < HALF)
    def _():
        out_ref[pl.ds(t, 1)] = pf[None]
        pb_store[pl.ds(s - HALF, 1)] = pb[None]

    @pl.when(t >= HALF)
    def _():
        prior_pf = out_ref[pl.ds(s, 1)][0]
        out_ref[pl.ds(s, 1)] = _sig(prior_pf + pb + fcb_ref[...])[None]
        stored_pb = pb_store[pl.ds(t - HALF, 1)][0]
        out_ref[pl.ds(t, 1)] = _sig(pf + stored_pb + fcb_ref[...])[None]


def kernel(emb, w_ih_f, w_hh_f, b_ih_f, b_hh_f, w_ih_b, w_hh_b, b_ih_b,
           b_hh_b, fc_w, fc_b, x):
    embp = _tpad_call(emb.T)
    idx = x.astype(jnp.int32).T.reshape(NW, NCHUNK, CHUNK)
    e_tb = _gather_call(embp, idx).reshape(T, B, EMBP)

    zpad = jnp.zeros((EMBP - EMB, 4 * HID), jnp.float32)
    wih_f_t = jnp.concatenate([w_ih_f.T, zpad], axis=0)
    whh_f_t = w_hh_f.T
    bias_f = (b_ih_f + b_hh_f).reshape(1, 4 * HID)
    wih_b_t = jnp.concatenate([w_ih_b.T, zpad], axis=0)
    whh_b_t = w_hh_b.T
    bias_b = (b_ih_b + b_hh_b).reshape(1, 4 * HID)
    fcw_t = fc_w.T
    fcw_f = fcw_t[:HID]
    fcw_b = fcw_t[HID:]
    fcb = fc_b.reshape(1, TAGS)

    def whole(shape):
        return pl.BlockSpec(shape, lambda t, _n=len(shape): (0,) * _n)

    out = pl.pallas_call(
        _bilstm_body,
        grid=(T,),
        in_specs=[
            pl.BlockSpec((1, B, EMBP), lambda t: (t, 0, 0)),
            pl.BlockSpec((1, B, EMBP), lambda t: (T - 1 - t, 0, 0)),
            whole((EMBP, 4 * HID)),
            whole((HID, 4 * HID)),
            whole((1, 4 * HID)),
            whole((EMBP, 4 * HID)),
            whole((HID, 4 * HID)),
            whole((1, 4 * HID)),
            whole((HID, TAGS)),
            whole((HID, TAGS)),
            whole((1, TAGS)),
        ],
        out_specs=whole((T, B, TAGS)),
        out_shape=jax.ShapeDtypeStruct((T, B, TAGS), jnp.float32),
        scratch_shapes=[
            pltpu.VMEM((B, HID), jnp.float32),
            pltpu.VMEM((B, HID), jnp.float32),
            pltpu.VMEM((B, HID), jnp.float32),
            pltpu.VMEM((B, HID), jnp.float32),
            pltpu.VMEM((HALF, B, TAGS), jnp.float32),
        ],
    )(e_tb, e_tb, wih_f_t, whh_f_t, bias_f, wih_b_t, whh_b_t, bias_b,
      fcw_f, fcw_b, fcb)

    return jnp.transpose(out, (1, 0, 2))


# speedup vs baseline: 1.2712x; 1.0050x over previous
import jax
import jax.numpy as jnp
from jax import lax
from jax.experimental import pallas as pl
from jax.experimental.pallas import tpu as pltpu
from jax.experimental.pallas import tpu_sc as plsc

VOCAB = 100000
EMB = 64
EMBP = 128
HID = 128
TAGS = 64
B = 1024
T = 50
HALF = T // 2

NC = 2
NS = 16
NW = NC * NS
BT = B * T
ROWS_PER_W = BT // NW
CHUNK = 80
NCHUNK = ROWS_PER_W // CHUNK
NBUF = 10

TBLK = 12800
TGRID = -(-VOCAB // TBLK)


def _tpad_body(et_ref, dst_ref):
    dst_ref[:, :EMB] = et_ref[...].T
    dst_ref[:, EMB:] = jnp.zeros((TBLK, EMBP - EMB), jnp.float32)


def _tpad_call(et):
    return pl.pallas_call(
        _tpad_body,
        grid=(TGRID,),
        in_specs=[pl.BlockSpec((EMB, TBLK), lambda i: (0, i))],
        out_specs=pl.BlockSpec((TBLK, EMBP), lambda i: (i, 0)),
        out_shape=jax.ShapeDtypeStruct((VOCAB, EMBP), jnp.float32),
    )(et)


def _gather_body(emb_hbm, idx_hbm, out_hbm, idx_v, rows_v, gsem, osem):
    wid = lax.axis_index("s") * NC + lax.axis_index("c")
    base = wid * ROWS_PER_W
    pltpu.sync_copy(idx_hbm.at[wid], idx_v)
    gathers = [None] * NCHUNK
    outs = [None] * NCHUNK
    for ci in range(NBUF):
        gathers[ci] = pltpu.async_copy(
            emb_hbm.at[idx_v.at[ci]], rows_v.at[ci], gsem
        )
    for ci in range(NCHUNK):
        gathers[ci].wait()
        outs[ci] = pltpu.async_copy(
            rows_v.at[ci % NBUF],
            out_hbm.at[pl.ds(base + ci * CHUNK, CHUNK)],
            osem,
        )
        nxt = ci + NBUF
        if nxt < NCHUNK:
            outs[ci].wait()
            gathers[nxt] = pltpu.async_copy(
                emb_hbm.at[idx_v.at[nxt]], rows_v.at[ci % NBUF], gsem
            )
    for ci in range(NCHUNK - NBUF, NCHUNK):
        outs[ci].wait()


def _gather_call(embp, idx):
    return pl.kernel(
        _gather_body,
        out_type=jax.ShapeDtypeStruct((BT, EMBP), jnp.float32),
        mesh=plsc.VectorSubcoreMesh(
            core_axis_name="c", subcore_axis_name="s",
            num_cores=NC, num_subcores=NS,
        ),
        scratch_types=[
            pltpu.VMEM((NCHUNK, CHUNK), jnp.int32),
            pltpu.VMEM((NBUF, CHUNK, EMBP), jnp.float32),
            pltpu.SemaphoreType.DMA,
            pltpu.SemaphoreType.DMA,
        ],
        compiler_params=pltpu.CompilerParams(use_tc_tiling_on_sc=False),
    )(embp, idx)


def _sig(x):
    return 0.5 * jnp.tanh(0.5 * x) + 0.5


def _lstm_step(first, e, h_ref, c_ref, wih_ref, whh_ref, bias_ref):
    hp = jnp.where(first, 0.0, h_ref[...])
    cp = jnp.where(first, 0.0, c_ref[...])
    g = jnp.dot(e, wih_ref[...], preferred_element_type=jnp.float32)
    g += jnp.dot(hp, whh_ref[...], preferred_element_type=jnp.float32)
    g += bias_ref[...]
    i = _sig(g[:, :HID])
    f = _sig(g[:, HID:2 * HID])
    gg = jnp.tanh(g[:, 2 * HID:3 * HID])
    o = _sig(g[:, 3 * HID:])
    c2 = f * cp + i * gg
    h2 = o * jnp.tanh(c2)
    h_ref[...] = h2
    c_ref[...] = c2
    return h2


def _bilstm_substep(t, s, ef, eb, wih_f, whh_f, bias_f, wih_b, whh_b,
                    bias_b, fcw_f, fcw_b, fcb_ref, out_ref,
                    hf_ref, cf_ref, hb_ref, cb_ref, pb_store):
    first = t == 0
    h2f = _lstm_step(first, ef, hf_ref, cf_ref, wih_f, whh_f, bias_f)
    h2b = _lstm_step(first, eb, hb_ref, cb_ref, wih_b, whh_b, bias_b)
    pf = jnp.dot(h2f, fcw_f[...], preferred_element_type=jnp.float32)
    pb = jnp.dot(h2b, fcw_b[...], preferred_element_type=jnp.float32)

    @pl.when(t < HALF)
    def _():
        out_ref[pl.ds(t, 1)] = pf[None]
        pb_store[pl.ds(s - HALF, 1)] = pb[None]

    @pl.when(t >= HALF)
    def _():
        prior_pf = out_ref[pl.ds(s, 1)][0]
        out_ref[pl.ds(s, 1)] = _sig(prior_pf + pb + fcb_ref[...])[None]
        stored_pb = pb_store[pl.ds(t - HALF, 1)][0]
        out_ref[pl.ds(t, 1)] = _sig(pf + stored_pb + fcb_ref[...])[None]


def _bilstm_body(ef_ref, eb_ref, wih_f, whh_f, bias_f, wih_b, whh_b, bias_b,
                 fcw_f, fcw_b, fcb_ref, out_ref,
                 hf_ref, cf_ref, hb_ref, cb_ref, pb_store):
    k = pl.program_id(0)
    rest = (wih_f, whh_f, bias_f, wih_b, whh_b, bias_b, fcw_f, fcw_b,
            fcb_ref, out_ref, hf_ref, cf_ref, hb_ref, cb_ref, pb_store)
    _bilstm_substep(2 * k, T - 1 - 2 * k, ef_ref[0], eb_ref[1], *rest)
    _bilstm_substep(2 * k + 1, T - 2 - 2 * k, ef_ref[1], eb_ref[0], *rest)


def kernel(emb, w_ih_f, w_hh_f, b_ih_f, b_hh_f, w_ih_b, w_hh_b, b_ih_b,
           b_hh_b, fc_w, fc_b, x):
    embp = _tpad_call(emb.T)
    idx = x.astype(jnp.int32).T.reshape(NW, NCHUNK, CHUNK)
    e_tb = _gather_call(embp, idx).reshape(T, B, EMBP)

    zpad = jnp.zeros((EMBP - EMB, 4 * HID), jnp.float32)
    wih_f_t = jnp.concatenate([w_ih_f.T, zpad], axis=0)
    whh_f_t = w_hh_f.T
    bias_f = (b_ih_f + b_hh_f).reshape(1, 4 * HID)
    wih_b_t = jnp.concatenate([w_ih_b.T, zpad], axis=0)
    whh_b_t = w_hh_b.T
    bias_b = (b_ih_b + b_hh_b).reshape(1, 4 * HID)
    fcw_t = fc_w.T
    fcw_f = fcw_t[:HID]
    fcw_b = fcw_t[HID:]
    fcb = fc_b.reshape(1, TAGS)

    def whole(shape):
        return pl.BlockSpec(shape, lambda t, _n=len(shape): (0,) * _n)

    out = pl.pallas_call(
        _bilstm_body,
        grid=(T // 2,),
        in_specs=[
            pl.BlockSpec((2, B, EMBP), lambda k: (k, 0, 0)),
            pl.BlockSpec((2, B, EMBP), lambda k: (T // 2 - 1 - k, 0, 0)),
            whole((EMBP, 4 * HID)),
            whole((HID, 4 * HID)),
            whole((1, 4 * HID)),
            whole((EMBP, 4 * HID)),
            whole((HID, 4 * HID)),
            whole((1, 4 * HID)),
            whole((HID, TAGS)),
            whole((HID, TAGS)),
            whole((1, TAGS)),
        ],
        out_specs=whole((T, B, TAGS)),
        out_shape=jax.ShapeDtypeStruct((T, B, TAGS), jnp.float32),
        scratch_shapes=[
            pltpu.VMEM((B, HID), jnp.float32),
            pltpu.VMEM((B, HID), jnp.float32),
            pltpu.VMEM((B, HID), jnp.float32),
            pltpu.VMEM((B, HID), jnp.float32),
            pltpu.VMEM((HALF, B, TAGS), jnp.float32),
        ],
    )(e_tb, e_tb, wih_f_t, whh_f_t, bias_f, wih_b_t, whh_b_t, bias_b,
      fcw_f, fcw_b, fcb)

    return jnp.transpose(out, (1, 0, 2))
